# baseline (speedup 1.0000x reference)
import jax
import jax.numpy as jnp
from jax.experimental import pallas as pl
from jax.experimental.pallas import tpu as pltpu

CAPACITY, GROUPS, HEAD_DIM = 8192, 8, 128
NEW_TOKENS = 32


def _body(len_ref, k_ref, v_ref, ak_ref, av_ref, ok_ref, ov_ref,
          sk, sv, sak, sav):
    cur = len_ref[0]
    cur = jnp.clip(cur, 0, CAPACITY - NEW_TOKENS)

    ck = pltpu.make_async_copy(k_ref, ok_ref, sk)
    cv = pltpu.make_async_copy(v_ref, ov_ref, sv)
    ck.start()
    cv.start()
    ck.wait()
    cv.wait()

    cak = pltpu.make_async_copy(ak_ref, ok_ref.at[pl.ds(cur, NEW_TOKENS)], sak)
    cav = pltpu.make_async_copy(av_ref, ov_ref.at[pl.ds(cur, NEW_TOKENS)], sav)
    cak.start()
    cav.start()
    cak.wait()
    cav.wait()


def kernel(keys, values, added_keys, added_values, current_length):
    num_added = added_keys.shape[0]
    cur1 = jnp.reshape(current_length, (1,)).astype(jnp.int32)
    out_shape = (
        jax.ShapeDtypeStruct(keys.shape, keys.dtype),
        jax.ShapeDtypeStruct(values.shape, values.dtype),
    )
    ok, ov = pl.pallas_call(
        _body,
        in_specs=[
            pl.BlockSpec(memory_space=pltpu.SMEM),
            pl.BlockSpec(memory_space=pltpu.MemorySpace.HBM),
            pl.BlockSpec(memory_space=pltpu.MemorySpace.HBM),
            pl.BlockSpec(memory_space=pltpu.MemorySpace.HBM),
            pl.BlockSpec(memory_space=pltpu.MemorySpace.HBM),
        ],
        out_specs=(
            pl.BlockSpec(memory_space=pltpu.MemorySpace.HBM),
            pl.BlockSpec(memory_space=pltpu.MemorySpace.HBM),
        ),
        out_shape=out_shape,
        scratch_shapes=[pltpu.SemaphoreType.DMA] * 4,
    )(cur1, keys, values, added_keys, added_values)
    return ok, ov, current_length + num_added


# speedup vs baseline: 1.0132x; 1.0132x over previous
import jax
import jax.numpy as jnp
from jax.experimental import pallas as pl
from jax.experimental.pallas import tpu as pltpu

CAPACITY, GROUPS, HEAD_DIM = 8192, 8, 128
NEW_TOKENS = 32


NCHUNK = 16
ROWS = CAPACITY // NCHUNK


def _body(len_ref, k_ref, v_ref, ak_ref, av_ref, ok_ref, ov_ref,
          sk, sv, sak, sav):
    cur = len_ref[0]
    cur = jnp.clip(cur, 0, CAPACITY - NEW_TOKENS)

    copies = []
    for i in range(NCHUNK):
        sl = pl.ds(i * ROWS, ROWS)
        copies.append(pltpu.make_async_copy(k_ref.at[sl], ok_ref.at[sl], sk))
        copies.append(pltpu.make_async_copy(v_ref.at[sl], ov_ref.at[sl], sv))
    for c in copies:
        c.start()
    for c in copies:
        c.wait()

    cak = pltpu.make_async_copy(ak_ref, ok_ref.at[pl.ds(cur, NEW_TOKENS)], sak)
    cav = pltpu.make_async_copy(av_ref, ov_ref.at[pl.ds(cur, NEW_TOKENS)], sav)
    cak.start()
    cav.start()
    cak.wait()
    cav.wait()


def kernel(keys, values, added_keys, added_values, current_length):
    num_added = added_keys.shape[0]
    cur1 = jnp.reshape(current_length, (1,)).astype(jnp.int32)
    out_shape = (
        jax.ShapeDtypeStruct(keys.shape, keys.dtype),
        jax.ShapeDtypeStruct(values.shape, values.dtype),
    )
    ok, ov = pl.pallas_call(
        _body,
        in_specs=[
            pl.BlockSpec(memory_space=pltpu.SMEM),
            pl.BlockSpec(memory_space=pltpu.MemorySpace.HBM),
            pl.BlockSpec(memory_space=pltpu.MemorySpace.HBM),
            pl.BlockSpec(memory_space=pltpu.MemorySpace.HBM),
            pl.BlockSpec(memory_space=pltpu.MemorySpace.HBM),
        ],
        out_specs=(
            pl.BlockSpec(memory_space=pltpu.MemorySpace.HBM),
            pl.BlockSpec(memory_space=pltpu.MemorySpace.HBM),
        ),
        out_shape=out_shape,
        scratch_shapes=[pltpu.SemaphoreType.DMA] * 4,
    )(cur1, keys, values, added_keys, added_values)
    return ok, ov, current_length + num_added


# speedup vs baseline: 12.2805x; 12.1207x over previous
import jax
import jax.numpy as jnp
from jax.experimental import pallas as pl
from jax.experimental.pallas import tpu as pltpu

CAPACITY, GROUPS, HEAD_DIM = 8192, 8, 128
NEW_TOKENS = 32
D = GROUPS * HEAD_DIM

S = 2
C = 512
NB = 2
RPS = CAPACITY // S
NCH = RPS // C


def _body(cur_ref, k_ref, v_ref, ak_ref, av_ref, ok_ref, ov_ref,
          buf, abuf, sem_in, sem_out, sem_a, sem_a2):
    cur = jnp.clip(cur_ref[0], 0, CAPACITY - NEW_TOKENS)
    cur = pl.multiple_of(cur, 8)

    chains = []
    for t, (src, dst) in enumerate(((k_ref, ok_ref), (v_ref, ov_ref))):
        for s in range(S):
            chains.append((t, s, src, dst))

    def in_copy(t, s, src, i):
        rows = pl.ds(s * RPS + i * C, C)
        return pltpu.make_async_copy(
            src.at[rows], buf.at[t, s, i % NB], sem_in.at[t, s, i % NB])

    def out_copy(t, s, dst, i):
        rows = pl.ds(s * RPS + i * C, C)
        return pltpu.make_async_copy(
            buf.at[t, s, i % NB], dst.at[rows], sem_out.at[t, s, i % NB])

    a_in = [pltpu.make_async_copy(ak_ref, abuf.at[0], sem_a.at[0]),
            pltpu.make_async_copy(av_ref, abuf.at[1], sem_a.at[1])]
    for c in a_in:
        c.start()

    for t, s, src, dst in chains:
        in_copy(t, s, src, 0).start()

    for i in range(NCH):
        for t, s, src, dst in chains:
            in_copy(t, s, src, i).wait()
            out_copy(t, s, dst, i).start()
        if i + 1 < NCH:
            for t, s, src, dst in chains:
                if i + 1 >= NB:
                    out_copy(t, s, dst, i + 1 - NB).wait()
                in_copy(t, s, src, i + 1).start()

    for t, s, src, dst in chains:
        for i in range(max(0, NCH - NB), NCH):
            out_copy(t, s, dst, i).wait()

    for c in a_in:
        c.wait()
    sl = pl.ds(cur, NEW_TOKENS)
    a_out = [pltpu.make_async_copy(abuf.at[0], ok_ref.at[sl], sem_a2.at[0]),
             pltpu.make_async_copy(abuf.at[1], ov_ref.at[sl], sem_a2.at[1])]
    for c in a_out:
        c.start()
    for c in a_out:
        c.wait()


def kernel(keys, values, added_keys, added_values, current_length):
    num_added = added_keys.shape[0]
    cur1 = jnp.reshape(current_length, (1,)).astype(jnp.int32)
    k2 = keys.reshape(CAPACITY, D)
    v2 = values.reshape(CAPACITY, D)
    ak2 = added_keys.reshape(NEW_TOKENS, D)
    av2 = added_values.reshape(NEW_TOKENS, D)

    out_shape = (
        jax.ShapeDtypeStruct((CAPACITY, D), keys.dtype),
        jax.ShapeDtypeStruct((CAPACITY, D), values.dtype),
    )
    hbm = pl.BlockSpec(memory_space=pltpu.MemorySpace.HBM)
    ok, ov = pl.pallas_call(
        _body,
        in_specs=[pl.BlockSpec(memory_space=pltpu.SMEM), hbm, hbm, hbm, hbm],
        out_specs=(hbm, hbm),
        out_shape=out_shape,
        scratch_shapes=[
            pltpu.VMEM((2, S, NB, C, D), jnp.float32),
            pltpu.VMEM((2, NEW_TOKENS, D), jnp.float32),
            pltpu.SemaphoreType.DMA((2, S, NB)),
            pltpu.SemaphoreType.DMA((2, S, NB)),
            pltpu.SemaphoreType.DMA((2,)),
            pltpu.SemaphoreType.DMA((2,)),
        ],
    )(cur1, k2, v2, ak2, av2)
    return (ok.reshape(CAPACITY, GROUPS, HEAD_DIM),
            ov.reshape(CAPACITY, GROUPS, HEAD_DIM),
            current_length + num_added)
